# baseline (speedup 1.0000x reference)
import functools

import jax
import jax.numpy as jnp
from jax import lax
from jax.experimental import pallas as pl
from jax.experimental.pallas import tpu as pltpu
from jax.experimental.pallas import tpu_sc as plsc

V = 1000
DEPTH = 4
B = 16
T = 512

NC = 2
NS = 16
NW = NC * NS

ROWS = B * T * DEPTH
RPW = ROWS // NW
K = 32
NCHUNK = RPW // K
LANES = 16
NVEC = RPW // LANES


def _tree_gather_kernel(table_hbm, idx_hbm, out_hbm,
                        idxrow_v, gidx_v, rows0_v, rows1_v, sem0, sem1):
    wid = lax.axis_index("s") * NC + lax.axis_index("c")
    r0 = wid * RPW
    b = wid // 2
    t_base = (wid % 2) * (T // 2)

    pltpu.sync_copy(idx_hbm.at[b], idxrow_v)

    def build(v, carry):
        lane = lax.iota(jnp.int32, LANES)
        zero = jnp.zeros((LANES,), jnp.int32)
        one = jnp.ones((LANES,), jnp.int32)
        d_lane = lane & jnp.full((LANES,), DEPTH - 1, jnp.int32)
        tq = lax.shift_right_logical(lane, jnp.full((LANES,), 2, jnp.int32))
        t = jnp.full((LANES,), t_base + v * DEPTH, jnp.int32) + tq
        src = t - d_lane
        tok = plsc.load_gather(idxrow_v, [jnp.maximum(src, zero)])
        tree_row = tok + d_lane * jnp.full((LANES,), V, jnp.int32) + jnp.full(
            (LANES,), 2 - V, jnp.int32)
        g = jnp.where(d_lane == zero, one,
                      jnp.where(src >= zero, tree_row, zero))
        gidx_v[v // (K // LANES), pl.ds((v % (K // LANES)) * LANES, LANES)] = g
        return carry

    lax.fori_loop(0, NVEC, build, 0)

    bufs = (rows0_v, rows1_v)
    sems = (sem0, sem1)

    def gather(c, j):
        return pltpu.make_async_copy(table_hbm.at[gidx_v.at[c]], bufs[j], sems[j])

    gather(0, 0).start()
    gather(1, 1).start()

    def body(c2, carry):
        for j in range(2):
            c = c2 + j
            gather(c, j).wait()
            pltpu.sync_copy(bufs[j], out_hbm.at[pl.ds(r0 + c * K, K)])

            @pl.when(c + 2 < NCHUNK)
            def _():
                gather(c + 2, j).start()
        return carry

    lax.fori_loop(0, NCHUNK // 2, lambda i, c: body(i * 2, c), 0)


@jax.jit
def kernel(idx, root_counts, tree_counts):
    aux = jnp.zeros((2, V), jnp.float32).at[1].set(root_counts)
    table = jnp.concatenate([aux, tree_counts.reshape(3 * V, V)], axis=0)

    mesh = plsc.VectorSubcoreMesh(core_axis_name="c", subcore_axis_name="s")
    run = functools.partial(
        pl.kernel,
        mesh=mesh,
        compiler_params=pltpu.CompilerParams(use_tc_tiling_on_sc=False, needs_layout_passes=False),
        out_type=jax.ShapeDtypeStruct((ROWS, V), jnp.float32),
        scratch_types=[
            pltpu.VMEM((T,), jnp.int32),
            pltpu.VMEM((NCHUNK, K), jnp.int32),
            pltpu.VMEM((K, V), jnp.float32),
            pltpu.VMEM((K, V), jnp.float32),
            pltpu.SemaphoreType.DMA,
            pltpu.SemaphoreType.DMA,
        ],
    )(_tree_gather_kernel)
    out = run(table, idx)
    return out.reshape(B, T, DEPTH, V)
